# baseline (speedup 1.0000x reference)
import jax
import jax.numpy as jnp
from jax.experimental import pallas as pl
from jax.experimental.pallas import tpu as pltpu

_TEMP = 0.1
_TINY = 1.1754944e-38
_EPS = 1e-12


def _loss_kernel(x_ref, labr_ref, laba_ref, m_ref, out_ref, rn_ref, acc_ref):
    i = pl.program_id(0)
    nsteps = pl.num_programs(0)

    @pl.when(i == 0)
    def _init():
        mm = m_ref[...]
        ones = jnp.ones((1, mm.shape[1]), jnp.float32)
        ssm = jax.lax.dot_general(
            mm * mm, ones, (((1,), (1,)), ((), ())),
            preferred_element_type=jnp.float32)
        rn = mm * jax.lax.rsqrt(jnp.maximum(ssm, _EPS * _EPS))
        rn_ref[...] = rn.astype(jnp.bfloat16)
        acc_ref[0] = 0.0
        acc_ref[1] = 0.0

    x = x_ref[...]
    onesf = jnp.ones((1, x.shape[1]), jnp.float32)
    ssx = jax.lax.dot_general(
        onesf, x * x, (((1,), (1,)), ((), ())),
        preferred_element_type=jnp.float32)
    xinv = jax.lax.rsqrt(jnp.maximum(ssx, _EPS * _EPS))
    xb = x.astype(jnp.bfloat16)
    inf = jnp.array(jnp.inf, jnp.bfloat16)
    sim = jax.lax.dot_general(
        rn_ref[...], xb, (((1,), (1,)), ((), ())),
        preferred_element_type=jnp.float32,
        ).astype(jnp.bfloat16)

    pos = laba_ref[...] == labr_ref[...]

    pos_min = jnp.min(jnp.where(pos, sim, inf), axis=0,
                      keepdims=True).astype(jnp.float32)
    neg_max32 = jnp.max(jnp.where(pos, -inf, sim), axis=0,
                        keepdims=True).astype(jnp.float32)

    valid = neg_max32 > -jnp.inf
    pos_sim = pos_min * xinv
    neg_sim = jnp.where(valid, neg_max32, 0.0) * xinv

    p = pos_sim / _TEMP
    n = neg_sim / _TEMP
    mx = jnp.maximum(p, n)
    num = jnp.exp(p - mx)
    den = jnp.exp(n - mx) + num
    losses = -jnp.log(num / den + _TINY)

    acc_ref[0] += jnp.sum(jnp.where(valid, losses, 0.0))
    acc_ref[1] += jnp.sum(jnp.where(valid, 1.0, 0.0))

    @pl.when(i == nsteps - 1)
    def _fin():
        loss = acc_ref[0] / jnp.maximum(acc_ref[1], 1.0)
        out_ref[...] = jnp.full((1, 1), loss, jnp.float32)


def kernel(inputs, labels, moco_inputs, emb_cq, label_cq, age_cq):
    B, F = inputs.shape
    TM = 1024
    lab_bf = jax.lax.bitcast_convert_type(
        (labels + 0x4000).astype(jnp.uint16), jnp.bfloat16)
    lab_col = lab_bf.reshape(B, 1)
    lab_row = lab_bf.reshape(1, B)
    out = pl.pallas_call(
        _loss_kernel,
        grid=(B // TM,),
        in_specs=[
            pl.BlockSpec((TM, F), lambda i: (i, 0)),
            pl.BlockSpec((1, TM), lambda i: (0, i)),
            pl.BlockSpec((B, 1), lambda i: (0, 0)),
            pl.BlockSpec((B, F), lambda i: (0, 0)),
        ],
        out_specs=pl.BlockSpec((1, 1), lambda i: (0, 0)),
        out_shape=jax.ShapeDtypeStruct((1, 1), jnp.float32),
        scratch_shapes=[
            pltpu.VMEM((B, F), jnp.bfloat16),
            pltpu.SMEM((2,), jnp.float32),
        ],
    )(inputs, lab_row, lab_col, moco_inputs)
    return out[0, 0]


# speedup vs baseline: 1.0331x; 1.0331x over previous
import jax
import jax.numpy as jnp
from jax.experimental import pallas as pl
from jax.experimental.pallas import tpu as pltpu

_TEMP = 0.1
_TINY = 1.1754944e-38
_EPS = 1e-12


def _loss_kernel(x_ref, labr_ref, laba_ref, m_ref, out_ref, rn_ref, acc_ref):
    i = pl.program_id(0)
    nsteps = pl.num_programs(0)

    @pl.when(i == 0)
    def _init():
        mm = m_ref[...]
        ones = jnp.ones((1, mm.shape[1]), jnp.float32)
        ssm = jax.lax.dot_general(
            mm * mm, ones, (((1,), (1,)), ((), ())),
            preferred_element_type=jnp.float32)
        rn = mm * jax.lax.rsqrt(jnp.maximum(ssm, _EPS * _EPS))
        rn_ref[...] = rn.astype(jnp.bfloat16)
        acc_ref[0] = 0.0
        acc_ref[1] = 0.0

    x = x_ref[...]
    onesf = jnp.ones((1, x.shape[1]), jnp.float32)
    ssx = jax.lax.dot_general(
        onesf, x * x, (((1,), (1,)), ((), ())),
        preferred_element_type=jnp.float32)
    xinv = jax.lax.rsqrt(jnp.maximum(ssx, _EPS * _EPS))
    xb = x.astype(jnp.bfloat16)
    inf = jnp.array(jnp.inf, jnp.bfloat16)
    sim = jax.lax.dot_general(
        rn_ref[...], xb, (((1,), (1,)), ((), ())),
        preferred_element_type=jnp.float32,
        ).astype(jnp.bfloat16)

    pos = laba_ref[...] == labr_ref[...]

    pos_min = jnp.min(jnp.where(pos, sim, inf), axis=0,
                      keepdims=True).astype(jnp.float32)
    neg_max32 = jnp.max(jnp.where(pos, -inf, sim), axis=0,
                        keepdims=True).astype(jnp.float32)

    valid = neg_max32 > -jnp.inf
    pos_sim = pos_min * xinv
    neg_sim = jnp.where(valid, neg_max32, 0.0) * xinv

    p = pos_sim / _TEMP
    n = neg_sim / _TEMP
    mx = jnp.maximum(p, n)
    num = jnp.exp(p - mx)
    den = jnp.exp(n - mx) + num
    losses = -jnp.log(num / den + _TINY)

    acc_ref[0] += jnp.sum(jnp.where(valid, losses, 0.0))
    acc_ref[1] += jnp.sum(jnp.where(valid, 1.0, 0.0))

    @pl.when(i == nsteps - 1)
    def _fin():
        loss = acc_ref[0] / jnp.maximum(acc_ref[1], 1.0)
        out_ref[...] = jnp.full((1, 1), loss, jnp.float32)


def kernel(inputs, labels, moco_inputs, emb_cq, label_cq, age_cq):
    B, F = inputs.shape
    TM = 4096
    lab_bf = jax.lax.bitcast_convert_type(
        (labels + 0x4000).astype(jnp.uint16), jnp.bfloat16)
    lab_col = lab_bf.reshape(B, 1)
    lab_row = lab_bf.reshape(1, B)
    out = pl.pallas_call(
        _loss_kernel,
        grid=(B // TM,),
        in_specs=[
            pl.BlockSpec((TM, F), lambda i: (i, 0)),
            pl.BlockSpec((1, TM), lambda i: (0, i)),
            pl.BlockSpec((B, 1), lambda i: (0, 0)),
            pl.BlockSpec((B, F), lambda i: (0, 0)),
        ],
        out_specs=pl.BlockSpec((1, 1), lambda i: (0, 0)),
        out_shape=jax.ShapeDtypeStruct((1, 1), jnp.float32),
        scratch_shapes=[
            pltpu.VMEM((B, F), jnp.bfloat16),
            pltpu.SMEM((2,), jnp.float32),
        ],
    )(inputs, lab_row, lab_col, moco_inputs)
    return out[0, 0]
